# initial kernel scaffold (unmeasured)
import jax
import jax.numpy as jnp
from jax import lax
from jax.experimental import pallas as pl
from jax.experimental.pallas import tpu as pltpu

T = 2048
V_SHARD = 16384
D = 1024


def kernel(ids, E):
    my_y = lax.axis_index("y")
    offset = my_y * V_SHARD
    local = ids - offset
    mask = (local >= 0) & (local < V_SHARD)
    safe = jnp.where(mask, local, 0)
    partial = jnp.where(mask[:, None], jnp.take(E, safe, axis=0), 0.0)

    def body(p_ref, out_ref, comm_ref, send_sem, recv_sem):
        my_x = lax.axis_index("x")
        my_y = lax.axis_index("y")
        my_z = lax.axis_index("z")
        nbr = (my_x, 1 - my_y, my_z)

        barrier_sem = pltpu.get_barrier_semaphore()
        pl.semaphore_signal(
            barrier_sem, inc=1, device_id=nbr,
            device_id_type=pl.DeviceIdType.MESH,
        )
        pl.semaphore_wait(barrier_sem, 1)

        rdma = pltpu.make_async_remote_copy(
            src_ref=p_ref,
            dst_ref=comm_ref,
            send_sem=send_sem,
            recv_sem=recv_sem,
            device_id=nbr,
            device_id_type=pl.DeviceIdType.MESH,
        )
        rdma.start()
        rdma.wait()

        out_ref[...] = p_ref[...] + comm_ref[...]

    return pl.pallas_call(
        body,
        out_shape=jax.ShapeDtypeStruct((T, D), jnp.float32),
        in_specs=[pl.BlockSpec(memory_space=pltpu.VMEM)],
        out_specs=pl.BlockSpec(memory_space=pltpu.VMEM),
        scratch_shapes=[
            pltpu.VMEM((T, D), jnp.float32),
            pltpu.SemaphoreType.DMA,
            pltpu.SemaphoreType.DMA,
        ],
        compiler_params=pltpu.CompilerParams(collective_id=0),
    )(partial)


# baseline (device time: 118963 ns/iter reference)
import jax
import jax.numpy as jnp
from jax import lax
from jax.experimental import pallas as pl
from jax.experimental.pallas import tpu as pltpu

T = 2048
V_SHARD = 16384
D = 1024
QT = T // 4


def kernel(ids, E):
    my_y = lax.axis_index("y")
    off = my_y * V_SHARD
    in_range = (ids >= off) & (ids < off + V_SHARD)
    maskf = in_range.astype(jnp.float32)[:, None]
    safe = jnp.clip(ids - off, 0, V_SHARD - 1).astype(jnp.int32)

    def body(ids_ref, mask_ref, e_ref, out_ref, g_ref, r_ref,
             gsem, ys, yr, xs, xr, zs, zr):
        my_x = lax.axis_index("x")
        my_y = lax.axis_index("y")
        my_z = lax.axis_index("z")
        q = 2 * my_z + my_x
        base = q * QT

        def gather_issue(t, carry):
            idx = ids_ref[base + t]
            pltpu.make_async_copy(
                e_ref.at[pl.ds(idx, 1), :],
                g_ref.at[pl.ds(t, 1), :],
                gsem,
            ).start()
            return carry
        lax.fori_loop(0, QT, gather_issue, 0)

        bsem = pltpu.get_barrier_semaphore()
        for nbr in ((1 - my_x, my_y, my_z),
                    (my_x, 1 - my_y, my_z),
                    (my_x, my_y, 1 - my_z)):
            pl.semaphore_signal(bsem, inc=1, device_id=nbr,
                                device_id_type=pl.DeviceIdType.MESH)
        pl.semaphore_wait(bsem, 3)

        def gather_wait(t, carry):
            pltpu.make_async_copy(
                e_ref.at[pl.ds(0, 1), :],
                g_ref.at[pl.ds(t, 1), :],
                gsem,
            ).wait()
            return carry
        lax.fori_loop(0, QT, gather_wait, 0)

        y_rdma = pltpu.make_async_remote_copy(
            src_ref=g_ref, dst_ref=r_ref, send_sem=ys, recv_sem=yr,
            device_id=(my_x, 1 - my_y, my_z),
            device_id_type=pl.DeviceIdType.MESH,
        )
        y_rdma.start()
        y_rdma.wait()

        m = mask_ref[pl.ds(base, QT), :]
        out_ref[pl.ds(base, QT), :] = (
            m * g_ref[...] + (1.0 - m) * r_ref[...]
        )

        x_rdma = pltpu.make_async_remote_copy(
            src_ref=out_ref.at[pl.ds(base, QT), :],
            dst_ref=out_ref.at[pl.ds(base, QT), :],
            send_sem=xs, recv_sem=xr,
            device_id=(1 - my_x, my_y, my_z),
            device_id_type=pl.DeviceIdType.MESH,
        )
        x_rdma.start()
        x_rdma.wait()

        half = my_z * (2 * QT)
        z_rdma = pltpu.make_async_remote_copy(
            src_ref=out_ref.at[pl.ds(half, 2 * QT), :],
            dst_ref=out_ref.at[pl.ds(half, 2 * QT), :],
            send_sem=zs, recv_sem=zr,
            device_id=(my_x, my_y, 1 - my_z),
            device_id_type=pl.DeviceIdType.MESH,
        )
        z_rdma.start()
        z_rdma.wait()

    return pl.pallas_call(
        body,
        out_shape=jax.ShapeDtypeStruct((T, D), jnp.float32),
        in_specs=[
            pl.BlockSpec(memory_space=pltpu.SMEM),
            pl.BlockSpec(memory_space=pltpu.VMEM),
            pl.BlockSpec(memory_space=pltpu.HBM),
        ],
        out_specs=pl.BlockSpec(memory_space=pltpu.VMEM),
        scratch_shapes=[
            pltpu.VMEM((QT, D), jnp.float32),
            pltpu.VMEM((QT, D), jnp.float32),
            pltpu.SemaphoreType.DMA,
            pltpu.SemaphoreType.DMA,
            pltpu.SemaphoreType.DMA,
            pltpu.SemaphoreType.DMA,
            pltpu.SemaphoreType.DMA,
            pltpu.SemaphoreType.DMA,
            pltpu.SemaphoreType.DMA,
        ],
        compiler_params=pltpu.CompilerParams(collective_id=0),
    )(safe, maskf, E)


# device time: 65763 ns/iter; 1.8090x vs baseline; 1.8090x over previous
import jax
import jax.numpy as jnp
from jax import lax
from jax.experimental import pallas as pl
from jax.experimental.pallas import tpu as pltpu

T = 2048
V_SHARD = 16384
D = 1024
QT = T // 4
C = 4
CT = QT // C
H = CT // 2

MESH = pl.DeviceIdType.MESH


def kernel(ids, E):
    my_y = lax.axis_index("y")
    off = my_y * V_SHARD
    in_range = (ids >= off) & (ids < off + V_SHARD)
    maskf = in_range.astype(jnp.float32)[:, None]
    safe = jnp.clip(ids - off, 0, V_SHARD - 1).astype(jnp.int32)

    def body(ids_ref, mask_ref, e_ref, out_ref, g_ref, r_ref,
             gsem, ys, yr, xs, xr, zs, zr, zfs, zdr, xfs, xdr):
        my_x = lax.axis_index("x")
        my_y = lax.axis_index("y")
        my_z = lax.axis_index("z")
        x_nbr = (1 - my_x, my_y, my_z)
        y_nbr = (my_x, 1 - my_y, my_z)
        z_nbr = (my_x, my_y, 1 - my_z)

        base = (2 * my_z + my_x) * QT
        xq_base = (2 * my_z + (1 - my_x)) * QT
        zq_base = (2 * (1 - my_z) + my_x) * QT
        dq_base = (2 * (1 - my_z) + (1 - my_x)) * QT

        def gather_issue(t, c):
            idx = ids_ref[base + c * CT + t]
            pltpu.make_async_copy(
                e_ref.at[pl.ds(idx, 1), :],
                g_ref.at[pl.ds(c * CT + t, 1), :],
                gsem.at[c],
            ).start()
            return c
        for c in range(C):
            lax.fori_loop(0, CT, gather_issue, c)

        bsem = pltpu.get_barrier_semaphore()
        for nbr in (x_nbr, y_nbr, z_nbr):
            pl.semaphore_signal(bsem, inc=1, device_id=nbr,
                                device_id_type=MESH)
        pl.semaphore_wait(bsem, 3)

        def chunk(ref, start, rows=CT):
            return ref.at[pl.ds(start, rows), :]

        y_rdma = []
        for c in range(C):
            def gather_wait(t, c):
                pltpu.make_async_copy(
                    e_ref.at[pl.ds(0, 1), :],
                    g_ref.at[pl.ds(c * CT + t, 1), :],
                    gsem.at[c],
                ).wait()
                return c
            lax.fori_loop(0, CT, gather_wait, c)
            r = pltpu.make_async_remote_copy(
                src_ref=chunk(g_ref, c * CT), dst_ref=chunk(r_ref, c * CT),
                send_sem=ys.at[c], recv_sem=yr.at[c],
                device_id=y_nbr, device_id_type=MESH,
            )
            r.start()
            y_rdma.append(r)

        x_rdma, zown_rdma = [], []
        for c in range(C):
            y_rdma[c].wait_recv()
            m = mask_ref[pl.ds(base + c * CT, CT), :]
            chunk(out_ref, base + c * CT)[...] = (
                m * chunk(g_ref, c * CT)[...]
                + (1.0 - m) * chunk(r_ref, c * CT)[...]
            )
            rx = pltpu.make_async_remote_copy(
                src_ref=chunk(out_ref, base + c * CT),
                dst_ref=chunk(out_ref, base + c * CT),
                send_sem=xs.at[c], recv_sem=xr.at[c],
                device_id=x_nbr, device_id_type=MESH,
            )
            rx.start()
            rz = pltpu.make_async_remote_copy(
                src_ref=chunk(out_ref, base + c * CT),
                dst_ref=chunk(out_ref, base + c * CT),
                send_sem=zs.at[c], recv_sem=zr.at[c],
                device_id=z_nbr, device_id_type=MESH,
            )
            rz.start()
            x_rdma.append(rx)
            zown_rdma.append(rz)

        zfwd_rdma = []
        for c in range(C):
            x_rdma[c].wait_recv()
            r = pltpu.make_async_remote_copy(
                src_ref=chunk(out_ref, xq_base + c * CT, H),
                dst_ref=chunk(out_ref, xq_base + c * CT, H),
                send_sem=zfs.at[c], recv_sem=zdr.at[c],
                device_id=z_nbr, device_id_type=MESH,
            )
            r.start()
            zfwd_rdma.append(r)

        xfwd_rdma = []
        for c in range(C):
            zown_rdma[c].wait_recv()
            r = pltpu.make_async_remote_copy(
                src_ref=chunk(out_ref, zq_base + c * CT + H, H),
                dst_ref=chunk(out_ref, zq_base + c * CT + H, H),
                send_sem=xfs.at[c], recv_sem=xdr.at[c],
                device_id=x_nbr, device_id_type=MESH,
            )
            r.start()
            xfwd_rdma.append(r)

        for c in range(C):
            pltpu.make_async_remote_copy(
                src_ref=chunk(out_ref, base + c * CT, H),
                dst_ref=chunk(out_ref, dq_base + c * CT, H),
                send_sem=zfs.at[c], recv_sem=zdr.at[c],
                device_id=z_nbr, device_id_type=MESH,
            ).wait_recv()
            pltpu.make_async_remote_copy(
                src_ref=chunk(out_ref, base + c * CT, H),
                dst_ref=chunk(out_ref, dq_base + c * CT + H, H),
                send_sem=xfs.at[c], recv_sem=xdr.at[c],
                device_id=x_nbr, device_id_type=MESH,
            ).wait_recv()
        for c in range(C):
            y_rdma[c].wait_send()
            x_rdma[c].wait_send()
            zown_rdma[c].wait_send()
            zfwd_rdma[c].wait_send()
            xfwd_rdma[c].wait_send()

    dma_c = pltpu.SemaphoreType.DMA((C,))
    return pl.pallas_call(
        body,
        out_shape=jax.ShapeDtypeStruct((T, D), jnp.float32),
        in_specs=[
            pl.BlockSpec(memory_space=pltpu.SMEM),
            pl.BlockSpec(memory_space=pltpu.VMEM),
            pl.BlockSpec(memory_space=pltpu.HBM),
        ],
        out_specs=pl.BlockSpec(memory_space=pltpu.VMEM),
        scratch_shapes=[
            pltpu.VMEM((QT, D), jnp.float32),
            pltpu.VMEM((QT, D), jnp.float32),
            dma_c,
            dma_c, dma_c,
            dma_c, dma_c,
            dma_c, dma_c,
            dma_c, dma_c,
            dma_c, dma_c,
        ],
        compiler_params=pltpu.CompilerParams(collective_id=0),
    )(safe, maskf, E)


# device time: 62905 ns/iter; 1.8912x vs baseline; 1.0454x over previous
import jax
import jax.numpy as jnp
from jax import lax
from jax.experimental import pallas as pl
from jax.experimental.pallas import tpu as pltpu

T = 2048
V_SHARD = 16384
D = 1024
QT = T // 4
C = 8
CT = QT // C
H = CT // 2

MESH = pl.DeviceIdType.MESH


def kernel(ids, E):
    my_y = lax.axis_index("y")
    off = my_y * V_SHARD
    lids = (ids - off).astype(jnp.int32)
    in_range = (lids >= 0) & (lids < V_SHARD)
    maskf = in_range.astype(jnp.float32)[:, None]

    def body(ids_ref, mask_ref, e_ref, out_ref, g_ref, r_ref,
             gsem, ys, yr, xs, xr, zs, zr, zfs, zdr, xfs, xdr):
        my_x = lax.axis_index("x")
        my_y = lax.axis_index("y")
        my_z = lax.axis_index("z")
        x_nbr = (1 - my_x, my_y, my_z)
        y_nbr = (my_x, 1 - my_y, my_z)
        z_nbr = (my_x, my_y, 1 - my_z)

        base = (2 * my_z + my_x) * QT
        xq_base = (2 * my_z + (1 - my_x)) * QT
        zq_base = (2 * (1 - my_z) + my_x) * QT
        dq_base = (2 * (1 - my_z) + (1 - my_x)) * QT

        n_dma = []
        for c in range(C):
            def gather_issue(t, n, c=c):
                lid = ids_ref[base + c * CT + t]
                valid = (lid >= 0) & (lid < V_SHARD)

                @pl.when(valid)
                def _():
                    pltpu.make_async_copy(
                        e_ref.at[pl.ds(lid, 1), :],
                        g_ref.at[pl.ds(c * CT + t, 1), :],
                        gsem.at[c],
                    ).start()
                return n + valid.astype(jnp.int32)
            n_dma.append(lax.fori_loop(0, CT, gather_issue, jnp.int32(0)))

        bsem = pltpu.get_barrier_semaphore()
        for nbr in (x_nbr, y_nbr, z_nbr):
            pl.semaphore_signal(bsem, inc=1, device_id=nbr,
                                device_id_type=MESH)
        pl.semaphore_wait(bsem, 3)

        def chunk(ref, start, rows=CT):
            return ref.at[pl.ds(start, rows), :]

        y_rdma = []
        for c in range(C):
            def gather_wait(t, c, c_=c):
                pltpu.make_async_copy(
                    e_ref.at[pl.ds(0, 1), :],
                    g_ref.at[pl.ds(c_ * CT, 1), :],
                    gsem.at[c_],
                ).wait()
                return c
            lax.fori_loop(0, n_dma[c], gather_wait, jnp.int32(0))
            r = pltpu.make_async_remote_copy(
                src_ref=chunk(g_ref, c * CT), dst_ref=chunk(r_ref, c * CT),
                send_sem=ys.at[c], recv_sem=yr.at[c],
                device_id=y_nbr, device_id_type=MESH,
            )
            r.start()
            y_rdma.append(r)

        x_rdma, zown_rdma = [], []
        for c in range(C):
            y_rdma[c].wait_recv()
            m = mask_ref[pl.ds(base + c * CT, CT), :]
            chunk(out_ref, base + c * CT)[...] = (
                m * chunk(g_ref, c * CT)[...]
                + (1.0 - m) * chunk(r_ref, c * CT)[...]
            )
            rx = pltpu.make_async_remote_copy(
                src_ref=chunk(out_ref, base + c * CT),
                dst_ref=chunk(out_ref, base + c * CT),
                send_sem=xs.at[c], recv_sem=xr.at[c],
                device_id=x_nbr, device_id_type=MESH,
            )
            rx.start()
            rz = pltpu.make_async_remote_copy(
                src_ref=chunk(out_ref, base + c * CT),
                dst_ref=chunk(out_ref, base + c * CT),
                send_sem=zs.at[c], recv_sem=zr.at[c],
                device_id=z_nbr, device_id_type=MESH,
            )
            rz.start()
            x_rdma.append(rx)
            zown_rdma.append(rz)

        zfwd_rdma = []
        for c in range(C):
            x_rdma[c].wait_recv()
            r = pltpu.make_async_remote_copy(
                src_ref=chunk(out_ref, xq_base + c * CT, H),
                dst_ref=chunk(out_ref, xq_base + c * CT, H),
                send_sem=zfs.at[c], recv_sem=zdr.at[c],
                device_id=z_nbr, device_id_type=MESH,
            )
            r.start()
            zfwd_rdma.append(r)

        xfwd_rdma = []
        for c in range(C):
            zown_rdma[c].wait_recv()
            r = pltpu.make_async_remote_copy(
                src_ref=chunk(out_ref, zq_base + c * CT + H, H),
                dst_ref=chunk(out_ref, zq_base + c * CT + H, H),
                send_sem=xfs.at[c], recv_sem=xdr.at[c],
                device_id=x_nbr, device_id_type=MESH,
            )
            r.start()
            xfwd_rdma.append(r)

        for c in range(C):
            pltpu.make_async_remote_copy(
                src_ref=chunk(out_ref, base + c * CT, H),
                dst_ref=chunk(out_ref, dq_base + c * CT, H),
                send_sem=zfs.at[c], recv_sem=zdr.at[c],
                device_id=z_nbr, device_id_type=MESH,
            ).wait_recv()
            pltpu.make_async_remote_copy(
                src_ref=chunk(out_ref, base + c * CT, H),
                dst_ref=chunk(out_ref, dq_base + c * CT + H, H),
                send_sem=xfs.at[c], recv_sem=xdr.at[c],
                device_id=x_nbr, device_id_type=MESH,
            ).wait_recv()
        for c in range(C):
            y_rdma[c].wait_send()
            x_rdma[c].wait_send()
            zown_rdma[c].wait_send()
            zfwd_rdma[c].wait_send()
            xfwd_rdma[c].wait_send()

    dma_c = pltpu.SemaphoreType.DMA((C,))
    return pl.pallas_call(
        body,
        out_shape=jax.ShapeDtypeStruct((T, D), jnp.float32),
        in_specs=[
            pl.BlockSpec(memory_space=pltpu.SMEM),
            pl.BlockSpec(memory_space=pltpu.VMEM),
            pl.BlockSpec(memory_space=pltpu.HBM),
        ],
        out_specs=pl.BlockSpec(memory_space=pltpu.VMEM),
        scratch_shapes=[
            pltpu.VMEM((QT, D), jnp.float32),
            pltpu.VMEM((QT, D), jnp.float32),
            dma_c,
            dma_c, dma_c,
            dma_c, dma_c,
            dma_c, dma_c,
            dma_c, dma_c,
            dma_c, dma_c,
        ],
        compiler_params=pltpu.CompilerParams(collective_id=0),
    )(lids, maskf, E)


# device time: 57545 ns/iter; 2.0673x vs baseline; 1.0931x over previous
import jax
import jax.numpy as jnp
from jax import lax
from jax.experimental import pallas as pl
from jax.experimental.pallas import tpu as pltpu

T = 2048
V_SHARD = 16384
D = 1024
QT = T // 4
C = 8
CT = QT // C
H = CT // 2

MESH = pl.DeviceIdType.MESH


def kernel(ids, E):
    my_y = lax.axis_index("y")
    off = my_y * V_SHARD
    lids = (ids - off).astype(jnp.int32)
    in_range = (lids >= 0) & (lids < V_SHARD)
    maskf = in_range.astype(jnp.float32)[:, None]

    def body(ids_ref, mask_ref, e_ref, out_ref, g_ref, r_ref,
             gsem, ys, yr, xs, xr, zs, zr, zfs, zdr, xfs, xdr):
        my_x = lax.axis_index("x")
        my_y = lax.axis_index("y")
        my_z = lax.axis_index("z")
        x_nbr = (1 - my_x, my_y, my_z)
        y_nbr = (my_x, 1 - my_y, my_z)
        z_nbr = (my_x, my_y, 1 - my_z)

        base = (2 * my_z + my_x) * QT
        xq_base = (2 * my_z + (1 - my_x)) * QT
        zq_base = (2 * (1 - my_z) + my_x) * QT
        dq_base = (2 * (1 - my_z) + (1 - my_x)) * QT

        bsem = pltpu.get_barrier_semaphore()
        for nbr in (x_nbr, y_nbr, z_nbr):
            pl.semaphore_signal(bsem, inc=1, device_id=nbr,
                                device_id_type=MESH)
        pl.semaphore_wait(bsem, 3)

        def chunk(ref, start, rows=CT):
            return ref.at[pl.ds(start, rows), :]

        def gather_issue_chunk(c):
            def gather_issue(t, n):
                lid = ids_ref[base + c * CT + t]
                valid = (lid >= 0) & (lid < V_SHARD)

                @pl.when(valid)
                def _():
                    pltpu.make_async_copy(
                        e_ref.at[pl.ds(lid, 1), :],
                        g_ref.at[pl.ds(c * CT + t, 1), :],
                        gsem.at[c],
                    ).start()
                return n + valid.astype(jnp.int32)
            return lax.fori_loop(0, CT, gather_issue, jnp.int32(0))

        def y_send_chunk(c, n_issued):
            def gather_wait(t, carry):
                pltpu.make_async_copy(
                    e_ref.at[pl.ds(0, 1), :],
                    g_ref.at[pl.ds(c * CT, 1), :],
                    gsem.at[c],
                ).wait()
                return carry
            lax.fori_loop(0, n_issued, gather_wait, jnp.int32(0))
            r = pltpu.make_async_remote_copy(
                src_ref=chunk(g_ref, c * CT), dst_ref=chunk(r_ref, c * CT),
                send_sem=ys.at[c], recv_sem=yr.at[c],
                device_id=y_nbr, device_id_type=MESH,
            )
            r.start()
            return r

        y_rdma = []
        n_prev = gather_issue_chunk(0)
        for c in range(C):
            n_cur = n_prev
            if c + 1 < C:
                n_prev = gather_issue_chunk(c + 1)
            y_rdma.append(y_send_chunk(c, n_cur))

        x_rdma, zown_rdma = [], []
        for c in range(C):
            y_rdma[c].wait_recv()
            m = mask_ref[pl.ds(base + c * CT, CT), :]
            chunk(out_ref, base + c * CT)[...] = (
                m * chunk(g_ref, c * CT)[...]
                + (1.0 - m) * chunk(r_ref, c * CT)[...]
            )
            rx = pltpu.make_async_remote_copy(
                src_ref=chunk(out_ref, base + c * CT),
                dst_ref=chunk(out_ref, base + c * CT),
                send_sem=xs.at[c], recv_sem=xr.at[c],
                device_id=x_nbr, device_id_type=MESH,
            )
            rx.start()
            rz = pltpu.make_async_remote_copy(
                src_ref=chunk(out_ref, base + c * CT),
                dst_ref=chunk(out_ref, base + c * CT),
                send_sem=zs.at[c], recv_sem=zr.at[c],
                device_id=z_nbr, device_id_type=MESH,
            )
            rz.start()
            x_rdma.append(rx)
            zown_rdma.append(rz)

        zfwd_rdma, xfwd_rdma = [], []
        for c in range(C):
            x_rdma[c].wait_recv()
            r = pltpu.make_async_remote_copy(
                src_ref=chunk(out_ref, xq_base + c * CT, H),
                dst_ref=chunk(out_ref, xq_base + c * CT, H),
                send_sem=zfs.at[c], recv_sem=zdr.at[c],
                device_id=z_nbr, device_id_type=MESH,
            )
            r.start()
            zfwd_rdma.append(r)
            zown_rdma[c].wait_recv()
            r = pltpu.make_async_remote_copy(
                src_ref=chunk(out_ref, zq_base + c * CT + H, H),
                dst_ref=chunk(out_ref, zq_base + c * CT + H, H),
                send_sem=xfs.at[c], recv_sem=xdr.at[c],
                device_id=x_nbr, device_id_type=MESH,
            )
            r.start()
            xfwd_rdma.append(r)

        for c in range(C):
            pltpu.make_async_remote_copy(
                src_ref=chunk(out_ref, base + c * CT, H),
                dst_ref=chunk(out_ref, dq_base + c * CT, H),
                send_sem=zfs.at[c], recv_sem=zdr.at[c],
                device_id=z_nbr, device_id_type=MESH,
            ).wait_recv()
            pltpu.make_async_remote_copy(
                src_ref=chunk(out_ref, base + c * CT, H),
                dst_ref=chunk(out_ref, dq_base + c * CT + H, H),
                send_sem=xfs.at[c], recv_sem=xdr.at[c],
                device_id=x_nbr, device_id_type=MESH,
            ).wait_recv()
        for c in range(C):
            y_rdma[c].wait_send()
            x_rdma[c].wait_send()
            zown_rdma[c].wait_send()
            zfwd_rdma[c].wait_send()
            xfwd_rdma[c].wait_send()

    dma_c = pltpu.SemaphoreType.DMA((C,))
    return pl.pallas_call(
        body,
        out_shape=jax.ShapeDtypeStruct((T, D), jnp.float32),
        in_specs=[
            pl.BlockSpec(memory_space=pltpu.SMEM),
            pl.BlockSpec(memory_space=pltpu.VMEM),
            pl.BlockSpec(memory_space=pltpu.HBM),
        ],
        out_specs=pl.BlockSpec(memory_space=pltpu.VMEM),
        scratch_shapes=[
            pltpu.VMEM((QT, D), jnp.float32),
            pltpu.VMEM((QT, D), jnp.float32),
            dma_c,
            dma_c, dma_c,
            dma_c, dma_c,
            dma_c, dma_c,
            dma_c, dma_c,
            dma_c, dma_c,
        ],
        compiler_params=pltpu.CompilerParams(collective_id=0),
    )(lids, maskf, E)


# device time: 50153 ns/iter; 2.3720x vs baseline; 1.1474x over previous
import jax
import jax.numpy as jnp
from jax import lax
from jax.experimental import pallas as pl
from jax.experimental.pallas import tpu as pltpu

T = 2048
V_SHARD = 16384
D = 1024
QT = T // 4
C = 16
CT = QT // C
H = CT // 2
U = 8
LY = 2
LF = 4

MESH = pl.DeviceIdType.MESH


def kernel(ids, E):
    my_y = lax.axis_index("y")
    off = my_y * V_SHARD
    lids = (ids - off).astype(jnp.int32)
    in_range = (lids >= 0) & (lids < V_SHARD)
    maskf = in_range.astype(jnp.float32)[:, None]

    def body(ids_ref, mask_ref, e_ref, out_ref, g_ref, r_ref,
             gsem, ys, yr, xs, xr, zs, zr, zfs, zdr, xfs, xdr):
        my_x = lax.axis_index("x")
        my_y = lax.axis_index("y")
        my_z = lax.axis_index("z")
        x_nbr = (1 - my_x, my_y, my_z)
        y_nbr = (my_x, 1 - my_y, my_z)
        z_nbr = (my_x, my_y, 1 - my_z)

        base = (2 * my_z + my_x) * QT
        xq_base = (2 * my_z + (1 - my_x)) * QT
        zq_base = (2 * (1 - my_z) + my_x) * QT
        dq_base = (2 * (1 - my_z) + (1 - my_x)) * QT

        bsem = pltpu.get_barrier_semaphore()
        for nbr in (x_nbr, y_nbr, z_nbr):
            pl.semaphore_signal(bsem, inc=1, device_id=nbr,
                                device_id_type=MESH)
        pl.semaphore_wait(bsem, 3)

        def chunk(ref, start, rows=CT):
            return ref.at[pl.ds(start, rows), :]

        def gather_issue_chunk(c):
            def gather_issue(i, n):
                for u in range(U):
                    t = i * U + u
                    lid = ids_ref[base + c * CT + t]
                    valid = (lid >= 0) & (lid < V_SHARD)

                    @pl.when(valid)
                    def _(t=t, lid=lid):
                        pltpu.make_async_copy(
                            e_ref.at[pl.ds(lid, 1), :],
                            g_ref.at[pl.ds(c * CT + t, 1), :],
                            gsem.at[c],
                        ).start()
                    n = n + valid.astype(jnp.int32)
                return n
            return lax.fori_loop(0, CT // U, gather_issue, jnp.int32(0))

        def y_send_chunk(c, n_issued):
            def gather_wait(t, carry):
                pltpu.make_async_copy(
                    e_ref.at[pl.ds(0, 1), :],
                    g_ref.at[pl.ds(c * CT, 1), :],
                    gsem.at[c],
                ).wait()
                return carry
            lax.fori_loop(0, n_issued, gather_wait, jnp.int32(0))
            r = pltpu.make_async_remote_copy(
                src_ref=chunk(g_ref, c * CT), dst_ref=chunk(r_ref, c * CT),
                send_sem=ys.at[c], recv_sem=yr.at[c],
                device_id=y_nbr, device_id_type=MESH,
            )
            r.start()
            return r

        y_rdma = [None] * C
        x_rdma = [None] * C
        zown_rdma = [None] * C
        zfwd_rdma = [None] * C
        xfwd_rdma = [None] * C

        def select_and_send(k):
            y_rdma[k].wait_recv()
            m = mask_ref[pl.ds(base + k * CT, CT), :]
            chunk(out_ref, base + k * CT)[...] = (
                m * chunk(g_ref, k * CT)[...]
                + (1.0 - m) * chunk(r_ref, k * CT)[...]
            )
            rx = pltpu.make_async_remote_copy(
                src_ref=chunk(out_ref, base + k * CT),
                dst_ref=chunk(out_ref, base + k * CT),
                send_sem=xs.at[k], recv_sem=xr.at[k],
                device_id=x_nbr, device_id_type=MESH,
            )
            rx.start()
            rz = pltpu.make_async_remote_copy(
                src_ref=chunk(out_ref, base + k * CT),
                dst_ref=chunk(out_ref, base + k * CT),
                send_sem=zs.at[k], recv_sem=zr.at[k],
                device_id=z_nbr, device_id_type=MESH,
            )
            rz.start()
            x_rdma[k], zown_rdma[k] = rx, rz

        def forward(k):
            x_rdma[k].wait_recv()
            rz = pltpu.make_async_remote_copy(
                src_ref=chunk(out_ref, xq_base + k * CT, H),
                dst_ref=chunk(out_ref, xq_base + k * CT, H),
                send_sem=zfs.at[k], recv_sem=zdr.at[k],
                device_id=z_nbr, device_id_type=MESH,
            )
            rz.start()
            zown_rdma[k].wait_recv()
            rx = pltpu.make_async_remote_copy(
                src_ref=chunk(out_ref, zq_base + k * CT + H, H),
                dst_ref=chunk(out_ref, zq_base + k * CT + H, H),
                send_sem=xfs.at[k], recv_sem=xdr.at[k],
                device_id=x_nbr, device_id_type=MESH,
            )
            rx.start()
            zfwd_rdma[k], xfwd_rdma[k] = rz, rx

        n_prev = gather_issue_chunk(0)
        for c in range(C):
            n_cur = n_prev
            if c + 1 < C:
                n_prev = gather_issue_chunk(c + 1)
            y_rdma[c] = y_send_chunk(c, n_cur)
            if c >= LY:
                select_and_send(c - LY)
            if c >= LF:
                forward(c - LF)

        for k in range(C - LY, C):
            select_and_send(k)
        for k in range(C - LF, C):
            forward(k)

        for k in range(C):
            pltpu.make_async_remote_copy(
                src_ref=chunk(out_ref, base + k * CT, H),
                dst_ref=chunk(out_ref, dq_base + k * CT, H),
                send_sem=zfs.at[k], recv_sem=zdr.at[k],
                device_id=z_nbr, device_id_type=MESH,
            ).wait_recv()
            pltpu.make_async_remote_copy(
                src_ref=chunk(out_ref, base + k * CT, H),
                dst_ref=chunk(out_ref, dq_base + k * CT + H, H),
                send_sem=xfs.at[k], recv_sem=xdr.at[k],
                device_id=x_nbr, device_id_type=MESH,
            ).wait_recv()
        for k in range(C):
            y_rdma[k].wait_send()
            x_rdma[k].wait_send()
            zown_rdma[k].wait_send()
            zfwd_rdma[k].wait_send()
            xfwd_rdma[k].wait_send()

    dma_c = pltpu.SemaphoreType.DMA((C,))
    return pl.pallas_call(
        body,
        out_shape=jax.ShapeDtypeStruct((T, D), jnp.float32),
        in_specs=[
            pl.BlockSpec(memory_space=pltpu.SMEM),
            pl.BlockSpec(memory_space=pltpu.VMEM),
            pl.BlockSpec(memory_space=pltpu.HBM),
        ],
        out_specs=pl.BlockSpec(memory_space=pltpu.VMEM),
        scratch_shapes=[
            pltpu.VMEM((QT, D), jnp.float32),
            pltpu.VMEM((QT, D), jnp.float32),
            dma_c,
            dma_c, dma_c,
            dma_c, dma_c,
            dma_c, dma_c,
            dma_c, dma_c,
            dma_c, dma_c,
        ],
        compiler_params=pltpu.CompilerParams(collective_id=0),
    )(lids, maskf, E)
